# Initial kernel scaffold; baseline (speedup 1.0000x reference)
import jax
import jax.numpy as jnp
from jax.experimental import pallas as pl


def kernel(feature_emb, kp_W, W_triu, W_tril):
    raise NotImplementedError("write your pallas kernel here")


# speedup vs baseline: 1.0211x; 1.0211x over previous
import functools

import jax
import jax.numpy as jnp
import numpy as np
from jax import lax
from jax.experimental import pallas as pl
from jax.experimental.pallas import tpu as pltpu

B, N, D, F = 4096, 64, 64, 4096
KPACK = 33 * N

_IU, _JU = np.triu_indices(N, 0)
_IL, _JL = np.tril_indices(N, 0)
_L_OF = np.zeros((N, N), dtype=np.int64)
_L_OF[_IL, _JL] = np.arange(_IL.size)
_TRIL_COL_FOR_UPPER = _L_OF[_JU, _IU]

_KPOS = np.where(
    _IU <= 31, 64 * _IU + (_JU - _IU),
    np.where(_JU > _IU, 64 * (63 - _IU) + _JU, 2048 + _IU))


def _pack_triu(m, bb):
    ii = lax.broadcasted_iota(jnp.int32, (bb, N, N), 1)
    ll = lax.broadcasted_iota(jnp.int32, (bb, N, N), 2)
    a = jnp.take_along_axis(m, (ii + ll) % N, axis=2)
    x8 = m.reshape(bb, 8, 8, N)
    xc = jnp.concatenate([x8[:, 7 - k:8 - k] for k in range(8)], axis=1)
    r7 = 7 - lax.broadcasted_iota(jnp.int32, (bb, 8, 8, N), 2)
    rev = jnp.take_along_axis(xc, r7, axis=2).reshape(bb, N, N)
    eye = (lax.broadcasted_iota(jnp.int32, (N, N), 0) ==
           lax.broadcasted_iota(jnp.int32, (N, N), 1)).astype(jnp.float32)
    dr = jnp.sum(m * eye[None], axis=1, keepdims=True)
    mask = ll[:, 0:32, :] <= 63 - ii[:, 0:32, :]
    p32 = jnp.where(mask, a[:, 0:32, :], rev[:, 0:32, :])
    packed = jnp.concatenate([p32, dr], axis=1)
    return packed.reshape(bb, KPACK)


def _pack_kernel(e_ref, aat_ref, p1_ref, p2_ref, *, bb):
    e = e_ref[...]
    ef = e.reshape(bb * N, D)
    t12 = jnp.dot(ef, aat_ref[...], preferred_element_type=jnp.float32)
    t12 = t12.reshape(bb, N, 2 * D)
    dn = (((2,), (2,)), ((0,), (0,)))
    m1 = lax.dot_general(t12[:, :, 0:D], e, dn,
                         preferred_element_type=jnp.float32)
    m2 = lax.dot_general(t12[:, :, D:2 * D], e, dn,
                         preferred_element_type=jnp.float32)
    p1_ref[...] = _pack_triu(m1, bb).astype(jnp.bfloat16)
    p2_ref[...] = _pack_triu(m2, bb).astype(jnp.bfloat16)


def _proj_kernel(p1_ref, p2_ref, wu_ref, wl_ref, o1_ref, o2_ref):
    o1_ref[...] = jnp.dot(p1_ref[...], wu_ref[...],
                          preferred_element_type=jnp.float32)
    o2_ref[...] = jnp.dot(p2_ref[...], wl_ref[...],
                          preferred_element_type=jnp.float32)


@jax.jit
def kernel(feature_emb, kp_W, W_triu, W_tril):
    bb = 128
    bbp, fb = 1024, 1024

    aat = jnp.concatenate([kp_W, kp_W.T], axis=1)

    p1, p2 = pl.pallas_call(
        functools.partial(_pack_kernel, bb=bb),
        grid=(B // bb,),
        in_specs=[
            pl.BlockSpec((bb, N, D), lambda i: (i, 0, 0)),
            pl.BlockSpec((D, 2 * D), lambda i: (0, 0)),
        ],
        out_specs=[
            pl.BlockSpec((bb, KPACK), lambda i: (i, 0)),
            pl.BlockSpec((bb, KPACK), lambda i: (i, 0)),
        ],
        out_shape=[
            jax.ShapeDtypeStruct((B, KPACK), jnp.bfloat16),
            jax.ShapeDtypeStruct((B, KPACK), jnp.bfloat16),
        ],
        compiler_params=pltpu.CompilerParams(
            dimension_semantics=("parallel",)),
    )(feature_emb, aat)

    wu = jnp.zeros((KPACK, F), jnp.float32).at[_KPOS].set(
        W_triu.T).astype(jnp.bfloat16)
    wl = jnp.zeros((KPACK, F), jnp.float32).at[_KPOS].set(
        W_tril[:, _TRIL_COL_FOR_UPPER].T).astype(jnp.bfloat16)

    v1, v2 = pl.pallas_call(
        _proj_kernel,
        grid=(B // bbp, F // fb),
        in_specs=[
            pl.BlockSpec((bbp, KPACK), lambda i, j: (i, 0)),
            pl.BlockSpec((bbp, KPACK), lambda i, j: (i, 0)),
            pl.BlockSpec((KPACK, fb), lambda i, j: (0, j)),
            pl.BlockSpec((KPACK, fb), lambda i, j: (0, j)),
        ],
        out_specs=[
            pl.BlockSpec((bbp, fb), lambda i, j: (i, j)),
            pl.BlockSpec((bbp, fb), lambda i, j: (i, j)),
        ],
        out_shape=[
            jax.ShapeDtypeStruct((B, F), jnp.float32),
            jax.ShapeDtypeStruct((B, F), jnp.float32),
        ],
        compiler_params=pltpu.CompilerParams(
            dimension_semantics=("parallel", "arbitrary")),
    )(p1, p2, wu, wl)

    embs_flatten = feature_emb.reshape(B, N * D)
    return (embs_flatten, v1, v2)


# speedup vs baseline: 1.0535x; 1.0317x over previous
import functools

import jax
import jax.numpy as jnp
import numpy as np
from jax import lax
from jax.experimental import pallas as pl
from jax.experimental.pallas import tpu as pltpu

B, N, D, F = 4096, 64, 64, 4096
KP = N * (N + 1) // 2
KPACK = 33 * N

_OFF_U = [64 * s - (s * (s - 1)) // 2 for s in range(N)]
_T = [s * (s + 1) // 2 for s in range(N)]


def _rev_rows(m, bb):
    x8 = m.reshape(bb, 8, 8, N)
    xc = jnp.concatenate([x8[:, 7 - k:8 - k] for k in range(8)], axis=1)
    r7 = 7 - lax.broadcasted_iota(jnp.int32, (bb, 8, 8, N), 2)
    return jnp.take_along_axis(xc, r7, axis=2).reshape(bb, N, N)


def _pack_both(m, bb):
    ii = lax.broadcasted_iota(jnp.int32, (bb, N, N), 1)
    ll = lax.broadcasted_iota(jnp.int32, (bb, N, N), 2)
    a1 = jnp.take_along_axis(m, (ii + ll) % N, axis=2)
    a2 = jnp.take_along_axis(m, (ii + ll + 1) % N, axis=2)
    rev = _rev_rows(m, bb)
    eye = (lax.broadcasted_iota(jnp.int32, (N, N), 0) ==
           lax.broadcasted_iota(jnp.int32, (N, N), 1)).astype(jnp.float32)
    dr = jnp.sum(m * eye[None], axis=1, keepdims=True)
    s32 = ii[:, 0:32, :]
    l32 = ll[:, 0:32, :]
    pu = jnp.where(l32 <= 63 - s32, a1[:, 0:32, :], rev[:, 0:32, :])
    plo = jnp.where(l32 < 63 - s32, rev[:, 0:32, :], a2[:, 0:32, :])
    packed_u = jnp.concatenate([pu, dr], axis=1).reshape(bb, KPACK)
    packed_l = jnp.concatenate([plo, dr], axis=1).reshape(bb, KPACK)
    return packed_u, packed_l


def _pack_kernel(e_ref, kpw_ref, p1_ref, p2_ref, *, bb):
    e = e_ref[...]
    ef = e.reshape(bb * N, D)
    t1 = jnp.dot(ef, kpw_ref[...], preferred_element_type=jnp.float32)
    dn = (((2,), (2,)), ((0,), (0,)))
    m = lax.dot_general(t1.reshape(bb, N, D), e, dn,
                        preferred_element_type=jnp.float32)
    packed_u, packed_l = _pack_both(m, bb)
    p1_ref[...] = packed_u.astype(jnp.bfloat16)
    p2_ref[...] = packed_l.astype(jnp.bfloat16)


def _relayout_wu(wt):
    pieces = []
    for s in range(32):
        o1 = _OFF_U[s]
        p1 = wt[:, o1:o1 + 64 - s]
        if s:
            o2 = _OFF_U[63 - s] + 1
            p1 = jnp.concatenate([p1, wt[:, o2:o2 + s]], axis=1)
        pieces.append(p1)
    kk = lax.broadcasted_iota(jnp.int32, (KP, 64), 0)
    cc = lax.broadcasted_iota(jnp.int32, (KP, 64), 1)
    sel = ((cc >= 32) & (kk == 64 * cc - (cc * (cc - 1)) // 2)
           ).astype(jnp.float32)
    diag = jnp.dot(wt, sel, preferred_element_type=jnp.float32)
    return jnp.concatenate(pieces + [diag], axis=1)


def _relayout_wl(wt):
    pieces = []
    for s in range(32):
        o1 = _T[63 - s]
        p1 = wt[:, o1:o1 + 63 - s]
        o2 = _T[s]
        pieces.append(jnp.concatenate([p1, wt[:, o2:o2 + s + 1]], axis=1))
    kk = lax.broadcasted_iota(jnp.int32, (KP, 64), 0)
    cc = lax.broadcasted_iota(jnp.int32, (KP, 64), 1)
    sel = ((cc >= 32) & (kk == (cc * (cc + 3)) // 2)).astype(jnp.float32)
    diag = jnp.dot(wt, sel, preferred_element_type=jnp.float32)
    return jnp.concatenate(pieces + [diag], axis=1)


def _proj_kernel(p1_ref, p2_ref, wu_ref, wl_ref, o1_ref, o2_ref,
                 wu_s, wl_s):
    i = pl.program_id(1)

    @pl.when(i == 0)
    def _():
        wu_s[...] = _relayout_wu(wu_ref[...]).astype(jnp.bfloat16)
        wl_s[...] = _relayout_wl(wl_ref[...]).astype(jnp.bfloat16)

    dn = (((1,), (1,)), ((), ()))
    o1_ref[...] = lax.dot_general(p1_ref[...], wu_s[...], dn,
                                  preferred_element_type=jnp.float32)
    o2_ref[...] = lax.dot_general(p2_ref[...], wl_s[...], dn,
                                  preferred_element_type=jnp.float32)


@jax.jit
def kernel(feature_emb, kp_W, W_triu, W_tril):
    bb = 128
    bbp, fb = 1024, 512

    p1, p2 = pl.pallas_call(
        functools.partial(_pack_kernel, bb=bb),
        grid=(B // bb,),
        in_specs=[
            pl.BlockSpec((bb, N, D), lambda i: (i, 0, 0)),
            pl.BlockSpec((D, D), lambda i: (0, 0)),
        ],
        out_specs=[
            pl.BlockSpec((bb, KPACK), lambda i: (i, 0)),
            pl.BlockSpec((bb, KPACK), lambda i: (i, 0)),
        ],
        out_shape=[
            jax.ShapeDtypeStruct((B, KPACK), jnp.bfloat16),
            jax.ShapeDtypeStruct((B, KPACK), jnp.bfloat16),
        ],
        compiler_params=pltpu.CompilerParams(
            dimension_semantics=("parallel",)),
    )(feature_emb, kp_W)

    v1, v2 = pl.pallas_call(
        _proj_kernel,
        grid=(F // fb, B // bbp),
        in_specs=[
            pl.BlockSpec((bbp, KPACK), lambda j, i: (i, 0)),
            pl.BlockSpec((bbp, KPACK), lambda j, i: (i, 0)),
            pl.BlockSpec((fb, KP), lambda j, i: (j, 0)),
            pl.BlockSpec((fb, KP), lambda j, i: (j, 0)),
        ],
        out_specs=[
            pl.BlockSpec((bbp, fb), lambda j, i: (i, j)),
            pl.BlockSpec((bbp, fb), lambda j, i: (i, j)),
        ],
        out_shape=[
            jax.ShapeDtypeStruct((B, F), jnp.float32),
            jax.ShapeDtypeStruct((B, F), jnp.float32),
        ],
        scratch_shapes=[
            pltpu.VMEM((fb, KPACK), jnp.bfloat16),
            pltpu.VMEM((fb, KPACK), jnp.bfloat16),
        ],
        compiler_params=pltpu.CompilerParams(
            dimension_semantics=("parallel", "arbitrary")),
    )(p1, p2, W_triu, W_tril)

    embs_flatten = feature_emb.reshape(B, N * D)
    return (embs_flatten, v1, v2)
